# baseline (speedup 1.0000x reference)
import functools

import jax
import jax.numpy as jnp
from jax import lax
from jax.experimental import pallas as pl
from jax.experimental.pallas import tpu as pltpu
from jax.experimental.pallas import tpu_sc as plsc

_MAX_DISTANCE = 128
_NUM_CORES = 2
_NUM_SUBCORES = 16
_LANES = 16


def _bias_body(q_len, k_len, heads, tab_hbm, out_hbm, t_vmem, n_vmem, sem):
    num_w = _NUM_CORES * _NUM_SUBCORES
    rows_per_w = q_len // num_w
    t_rows = 2 * _MAX_DISTANCE - 1
    n_rows = q_len + k_len - 1
    lo_base = q_len - _MAX_DISTANCE

    wid = lax.axis_index("s") * _NUM_CORES + lax.axis_index("c")

    pltpu.sync_copy(tab_hbm, t_vmem)

    hi_row = t_vmem[pl.ds((t_rows - 1) * heads, _LANES)]
    lo_row = t_vmem[pl.ds(0, _LANES)]

    def mid_body(r, carry):
        src = t_vmem[pl.ds(pl.multiple_of(r * heads, heads), _LANES)]
        dst = (lo_base + t_rows - 1) * heads - r * heads
        n_vmem[pl.ds(pl.multiple_of(dst, heads), _LANES)] = src
        return carry

    lax.fori_loop(0, t_rows, mid_body, 0)

    def fill_region(base_words, rows, row_vec):
        def body(it, carry):
            base = base_words + it * (8 * heads)
            for k in range(8):
                n_vmem[pl.ds(pl.multiple_of(base + k * heads, heads), _LANES)] = row_vec
            return carry
        lax.fori_loop(0, rows // 8, body, 0)

    fill_region(0, lo_base, hi_row)
    fill_region((lo_base + t_rows) * heads, n_rows - lo_base - t_rows,
                lo_row)

    row0 = wid * rows_per_w
    row_words = k_len * heads

    K = 8

    def emit_group(g, carry):
        i0 = row0 + g * K
        handles = []
        for k in range(K):
            i = i0 + k
            start = pl.multiple_of((q_len - 1 - i) * heads, heads)
            handles.append(pltpu.async_copy(
                n_vmem.at[pl.ds(start, row_words)],
                out_hbm.at[pl.ds(i * row_words, row_words)], sem))
        for h in handles:
            h.wait()
        return carry

    lax.fori_loop(0, rows_per_w // K, emit_group, 0)


def kernel(x, relative_attention_bias_table):
    q_len = x.shape[1]
    k_len = x.shape[1]
    t_rows, heads = relative_attention_bias_table.shape
    assert t_rows == 2 * _MAX_DISTANCE - 1 and heads == _LANES
    assert q_len % (_NUM_CORES * _NUM_SUBCORES * 8) == 0

    n_rows_padded = q_len + k_len
    mesh = plsc.VectorSubcoreMesh(core_axis_name="c", subcore_axis_name="s")
    grid_kernel = functools.partial(
        pl.kernel,
        out_type=jax.ShapeDtypeStruct((q_len * k_len * heads,), jnp.float32),
        mesh=mesh,
        scratch_types=[
            pltpu.VMEM((t_rows * heads,), jnp.float32),
            pltpu.VMEM((n_rows_padded * heads,), jnp.float32),
            pltpu.SemaphoreType.DMA,
        ],
    )(functools.partial(_bias_body, q_len, k_len, heads))

    out_flat = grid_kernel(relative_attention_bias_table.reshape(-1))
    return out_flat.reshape(q_len, k_len, heads)


# speedup vs baseline: 10.2528x; 10.2528x over previous
import functools

import jax
import jax.numpy as jnp
from jax import lax
from jax.experimental import pallas as pl
from jax.experimental.pallas import tpu as pltpu
from jax.experimental.pallas import tpu_sc as plsc

_MAX_DISTANCE = 128
_NUM_CORES = 2
_NUM_SUBCORES = 16
_LANES = 16

_NT_COLS = 4224
_PH_STRIDE = 4232


def _bias_body(q_len, heads, tab_hbm, out_hbm, t_vmem, row8, nt8, sem):
    num_w = _NUM_CORES * _NUM_SUBCORES
    rows_per_w = q_len // num_w
    t_rows = 2 * _MAX_DISTANCE - 1
    lo_base = q_len - _MAX_DISTANCE

    sid = lax.axis_index("s")
    cid = lax.axis_index("c")
    wid = sid * _NUM_CORES + cid

    pltpu.sync_copy(tab_hbm, t_vmem)

    hi_vec = plsc.load_gather(
        t_vmem, [jnp.full((_LANES,), (t_rows - 1) * heads + sid, jnp.int32)])
    lo_vec = plsc.load_gather(
        t_vmem, [jnp.full((_LANES,), sid, jnp.int32)])
    lane = lax.iota(jnp.int32, _LANES)

    for p in range(8):
        base = p * _PH_STRIDE

        def left_body(k, carry, base=base):
            off = base + k * 128
            for u in range(8):
                row8[pl.ds(off + u * _LANES, _LANES)] = hi_vec
            return carry

        lax.fori_loop(0, 15, left_body, 0)

        for c in range(16):
            t_idx = jnp.maximum(t_rows - 1 - (c * _LANES + lane), 0)
            vals = plsc.load_gather(t_vmem, [t_idx * heads + sid])
            row8[pl.ds(base + lo_base - p + c * _LANES, _LANES)] = vals

        rstart = base + lo_base + 256 - p

        def right_body(k, carry, rstart=rstart):
            off = rstart + k * 128
            for u in range(8):
                row8[pl.ds(off + u * _LANES, _LANES)] = lo_vec
            return carry

        lax.fori_loop(0, 16, right_body, 0)

    for p in range(8):
        pltpu.sync_copy(
            row8.at[pl.ds(p * _PH_STRIDE, _NT_COLS)], nt8.at[p, sid])

    plsc.subcore_barrier()

    row0 = wid * rows_per_w

    def emit(r, carry):
        i = row0 + r
        s = q_len - 1 - i
        p = lax.bitwise_and(s, 7)
        sal = s - p
        handles = []
        for ht in range(2):
            for jt in range(16):
                src = nt8.at[
                    p,
                    pl.ds(ht * 8, 8),
                    pl.ds(pl.multiple_of(sal + jt * 128, 8), 128)]
                dst = out_hbm.at[i, pl.ds(ht * 128 + jt * 8, 8), :]
                handles.append(pltpu.async_copy(src, dst, sem))
        for h_ in handles:
            h_.wait()
        return carry

    lax.fori_loop(0, rows_per_w, emit, 0)


def kernel(x, relative_attention_bias_table):
    q_len = x.shape[1]
    t_rows, heads = relative_attention_bias_table.shape
    assert t_rows == 2 * _MAX_DISTANCE - 1 and heads == _LANES
    assert q_len == 2048

    mesh = plsc.VectorSubcoreMesh(core_axis_name="c", subcore_axis_name="s")
    grid_kernel = functools.partial(
        pl.kernel,
        out_type=jax.ShapeDtypeStruct((q_len, 256, 128), jnp.float32),
        mesh=mesh,
        scratch_types=[
            pltpu.VMEM((t_rows * heads,), jnp.float32),
            pltpu.VMEM((8 * _PH_STRIDE + _LANES,), jnp.float32),
            pltpu.VMEM_SHARED((8, heads, _NT_COLS), jnp.float32),
            pltpu.SemaphoreType.DMA,
        ],
        compiler_params=pltpu.CompilerParams(
            use_tc_tiling_on_sc=False, needs_layout_passes=False),
    )(functools.partial(_bias_body, q_len, heads))

    out5 = grid_kernel(relative_attention_bias_table.reshape(-1))
    return (out5.reshape(q_len, 2, 16, 8, 128)
            .transpose(0, 2, 4, 1, 3)
            .reshape(q_len, q_len, heads))
